# Initial kernel scaffold; baseline (speedup 1.0000x reference)
import jax
import jax.numpy as jnp
from jax.experimental import pallas as pl


def kernel(x, weight, bias):
    raise NotImplementedError("write your pallas kernel here")


# speedup vs baseline: 1.0686x; 1.0686x over previous
import functools

import jax
import jax.numpy as jnp
from jax.experimental import pallas as pl
from jax.experimental.pallas import tpu as pltpu


def _tcn_kernel(cur_ref, halo_ref, w_ref, b_ref, o_ref, *, kernel_size, dilation):
    K = kernel_size
    pad = (K - 1) * dilation
    HL = halo_ref.shape[2]
    TT = cur_ref.shape[2]

    cur = cur_ref[0]

    halo_tail = halo_ref[0, :, HL - pad:]
    halo_tail = jnp.where(pl.program_id(1) == 0,
                          jnp.zeros_like(halo_tail), halo_tail)

    acc = jnp.dot(w_ref[K - 1], cur, preferred_element_type=jnp.float32)
    for k in range(K - 1):
        shift = (K - 1 - k) * dilation
        xs = jnp.concatenate(
            [halo_tail[:, pad - shift:], cur[:, :TT - shift]], axis=-1)
        acc = acc + jnp.dot(w_ref[k], xs, preferred_element_type=jnp.float32)

    o_ref[0] = (acc + b_ref[...]).astype(o_ref.dtype)


def kernel(x, weight, bias):
    B, C_in, T = x.shape
    C_out = weight.shape[0]
    K = weight.shape[2]
    dilation = 1
    pad = (K - 1) * dilation
    HL = 128
    assert pad <= HL

    if T <= 2048 or T % 2048 != 0:
        TT = T
    else:
        TT = 2048
    NT = T // TT
    halo_blocks_per_tile = max(TT // HL, 1)

    wt = jnp.transpose(weight, (2, 0, 1))
    b2 = bias.reshape(C_out, 1)

    kernel_fn = functools.partial(_tcn_kernel, kernel_size=K, dilation=dilation)

    return pl.pallas_call(
        kernel_fn,
        out_shape=jax.ShapeDtypeStruct((B, C_out, T), x.dtype),
        grid=(B, NT),
        in_specs=[
            pl.BlockSpec((1, C_in, TT), lambda b, t: (b, 0, t)),
            pl.BlockSpec(
                (1, C_in, HL),
                lambda b, t: (b, 0, jnp.maximum(t * halo_blocks_per_tile - 1, 0))),
            pl.BlockSpec((K, C_out, C_in), lambda b, t: (0, 0, 0)),
            pl.BlockSpec((C_out, 1), lambda b, t: (0, 0)),
        ],
        out_specs=pl.BlockSpec((1, C_out, TT), lambda b, t: (b, 0, t)),
        compiler_params=pltpu.CompilerParams(
            dimension_semantics=("parallel", "parallel")),
    )(x, x, wt, b2)


# speedup vs baseline: 1.3085x; 1.2245x over previous
import functools

import jax
import jax.numpy as jnp
from jax.experimental import pallas as pl
from jax.experimental.pallas import tpu as pltpu


def _tcn_kernel(cur_ref, halo_ref, w_ref, b_ref, o_ref, *, kernel_size, dilation):
    K = kernel_size
    pad = (K - 1) * dilation
    HL = halo_ref.shape[2]
    TT = cur_ref.shape[2]

    cur = cur_ref[0]

    halo_tail = halo_ref[0, :, HL - pad:]
    halo_tail = jnp.where(pl.program_id(1) == 0,
                          jnp.zeros_like(halo_tail), halo_tail)

    acc = jnp.dot(w_ref[K - 1], cur, preferred_element_type=jnp.float32)
    for k in range(K - 1):
        shift = (K - 1 - k) * dilation
        xs = jnp.concatenate(
            [halo_tail[:, pad - shift:], cur[:, :TT - shift]], axis=-1)
        acc = acc + jnp.dot(w_ref[k], xs, preferred_element_type=jnp.float32)

    o_ref[0] = (acc + b_ref[...]).astype(o_ref.dtype)


def kernel(x, weight, bias):
    B, C_in, T = x.shape
    C_out = weight.shape[0]
    K = weight.shape[2]
    dilation = 1
    pad = (K - 1) * dilation
    HL = 128
    assert pad <= HL

    TT = T
    NT = T // TT
    halo_blocks_per_tile = max(TT // HL, 1)

    wt = jnp.transpose(weight, (2, 0, 1))
    b2 = bias.reshape(C_out, 1)

    kernel_fn = functools.partial(_tcn_kernel, kernel_size=K, dilation=dilation)

    return pl.pallas_call(
        kernel_fn,
        out_shape=jax.ShapeDtypeStruct((B, C_out, T), x.dtype),
        grid=(B, NT),
        in_specs=[
            pl.BlockSpec((1, C_in, TT), lambda b, t: (b, 0, t)),
            pl.BlockSpec(
                (1, C_in, HL),
                lambda b, t: (b, 0, jnp.maximum(t * halo_blocks_per_tile - 1, 0))),
            pl.BlockSpec((K, C_out, C_in), lambda b, t: (0, 0, 0)),
            pl.BlockSpec((C_out, 1), lambda b, t: (0, 0)),
        ],
        out_specs=pl.BlockSpec((1, C_out, TT), lambda b, t: (b, 0, t)),
        compiler_params=pltpu.CompilerParams(
            dimension_semantics=("parallel", "parallel"),
            vmem_limit_bytes=100 * 1024 * 1024),
    )(x, x, wt, b2)


# speedup vs baseline: 1.3348x; 1.0201x over previous
import functools

import jax
import jax.numpy as jnp
from jax.experimental import pallas as pl
from jax.experimental.pallas import tpu as pltpu


def _tcn_kernel(cur_ref, w_ref, b_ref, o_ref, *, kernel_size, dilation):
    K = kernel_size
    TT = cur_ref.shape[2]
    C_in = cur_ref.shape[1]

    cur = cur_ref[0]

    acc = jnp.dot(w_ref[K - 1], cur, preferred_element_type=jnp.float32)
    for k in range(K - 1):
        shift = (K - 1 - k) * dilation
        xs = jnp.concatenate(
            [jnp.zeros((C_in, shift), cur.dtype), cur[:, :TT - shift]], axis=-1)
        acc = acc + jnp.dot(w_ref[k], xs, preferred_element_type=jnp.float32)

    o_ref[0] = (acc + b_ref[...]).astype(o_ref.dtype)


def kernel(x, weight, bias):
    B, C_in, T = x.shape
    C_out = weight.shape[0]
    K = weight.shape[2]
    dilation = 1

    wt = jnp.transpose(weight, (2, 0, 1))
    b2 = bias.reshape(C_out, 1)

    kernel_fn = functools.partial(_tcn_kernel, kernel_size=K, dilation=dilation)

    return pl.pallas_call(
        kernel_fn,
        out_shape=jax.ShapeDtypeStruct((B, C_out, T), x.dtype),
        grid=(B,),
        in_specs=[
            pl.BlockSpec((1, C_in, T), lambda b: (b, 0, 0)),
            pl.BlockSpec((K, C_out, C_in), lambda b: (0, 0, 0)),
            pl.BlockSpec((C_out, 1), lambda b: (0, 0)),
        ],
        out_specs=pl.BlockSpec((1, C_out, T), lambda b: (b, 0, 0)),
        compiler_params=pltpu.CompilerParams(
            dimension_semantics=("parallel",),
            vmem_limit_bytes=100 * 1024 * 1024),
    )(x, wt, b2)
